# baseline (speedup 1.0000x reference)
import functools
import math

import jax
import jax.numpy as jnp
from jax import lax
from jax.experimental import pallas as pl
from jax.experimental.pallas import tpu as pltpu
from jax.experimental.pallas import tpu_sc as plsc

_SMOOTH = 0.1
_PAD = 0
_BV = 4096
_LW = 512
_NW = 32
_SC_ROWS = 256
_SC_COLS = 99968
_SEG = 2048


def _sc_rowsum_kernel(p, segs, offs, *, n_tc, vocab):
    rpt = _SC_ROWS // _NW
    nfull, rem = divmod(_SC_COLS, _SEG)
    chunk_plan = [(k * _SEG, _SEG) for k in range(nfull)]
    if rem:
        chunk_plan.append((nfull * _SEG, rem))
    mesh = plsc.VectorSubcoreMesh(core_axis_name="c", subcore_axis_name="s")

    @functools.partial(
        pl.kernel,
        mesh=mesh,
        out_type=(
            jax.ShapeDtypeStruct((_SC_ROWS, 16), jnp.float32),
            jax.ShapeDtypeStruct((_SC_ROWS, 16), jnp.float32),
            jax.ShapeDtypeStruct((_SC_ROWS, 16), jnp.float32),
        ),
        scratch_types=[
            pltpu.VMEM((rpt, _SEG), jnp.float32),
            pltpu.VMEM((rpt, _SEG), jnp.float32),
            pltpu.VMEM((rpt, 16), jnp.int32),
            pltpu.VMEM((rpt, 16), jnp.int32),
            pltpu.VMEM((rpt, 16), jnp.float32),
            pltpu.SemaphoreType.DMA,
            pltpu.SemaphoreType.DMA,
        ],
    )
    def body(p_hbm, segs_hbm, offs_hbm, sums_hbm, pt_hbm, p0_hbm,
             buf0, buf1, segsp_ref, offsp_ref, stage, sem0, sem1):
        w = lax.axis_index("s") * 2 + lax.axis_index("c")
        row0 = n_tc + w * rpt
        pltpu.sync_copy(segs_hbm.at[w], segsp_ref)
        pltpu.sync_copy(offs_hbm.at[w], offsp_ref)
        lane = lax.iota(jnp.int32, 16)
        zero16 = jnp.zeros((16,), jnp.float32)

        bufs = (buf0, buf1)
        sems = (sem0, sem1)

        def chunk_copy(u):
            c0, cw = chunk_plan[u]
            dst = bufs[u % 2]
            if cw != _SEG:
                dst = dst.at[:, pl.ds(0, cw)]
            return pltpu.async_copy(
                p_hbm.at[pl.ds(row0, rpt), pl.ds(c0, cw)], dst, sems[u % 2])

        handles = {0: chunk_copy(0)}
        accs = (zero16,) * rpt
        ptaccs = (zero16,) * rpt
        p0accs = (zero16,) * rpt
        for u, (c0, cw) in enumerate(chunk_plan):
            if u + 1 < len(chunk_plan):
                handles[u + 1] = chunk_copy(u + 1)
            handles.pop(u).wait()
            buf = bufs[u % 2]
            segsp = [segsp_ref[r, :] for r in range(rpt)]
            offsp = [offsp_ref[r, :] for r in range(rpt)]

            def acc_step(j, carry, _buf=buf, _segsp=segsp, _offsp=offsp,
                         _u=u, _first=(u == 0)):
                a, pa, za = carry
                jvec = j * 16 + lane
                a_n, pa_n, za_n = [], [], []
                for r in range(rpt):
                    x = _buf[r, pl.ds(j * 16, 16)]
                    a_n.append(a[r] + x)
                    hitm = (_segsp[r] == _u) & (jvec == _offsp[r])
                    pa_n.append(pa[r] + jnp.where(hitm, x, zero16))
                    if _first:
                        za_n.append(za[r] + jnp.where(jvec == 0, x, zero16))
                    else:
                        za_n.append(za[r])
                return tuple(a_n), tuple(pa_n), tuple(za_n)

            accs, ptaccs, p0accs = lax.fori_loop(
                0, cw // 16, acc_step, (accs, ptaccs, p0accs), unroll=2)

        for vecs, dst in ((accs, sums_hbm), (ptaccs, pt_hbm),
                          (p0accs, p0_hbm)):
            for r in range(rpt):
                stage[r, :] = vecs[r]
            pltpu.sync_copy(stage, dst.at[pl.ds(w * rpt, rpt)])

    return body(p, segs, offs)


def _tc_body(p_ref, ptail_ref, t_ref, tsc_ref,
             out_ref, tails_ref, tailpt_ref,
             acc_ref, pt_ref, p0_ref, *, n_rows, n_sc, vocab):
    i = pl.program_id(0)
    nb = pl.num_programs(0)

    @pl.when(i == 0)
    def _init():
        acc_ref[...] = jnp.zeros_like(acc_ref)
        pt_ref[...] = jnp.zeros_like(pt_ref)
        p0_ref[...] = p_ref[:, 0:1]
        colt = _SC_COLS + jax.lax.broadcasted_iota(jnp.int32, (n_sc, 128), 1)
        xt = jnp.where(colt < vocab, ptail_ref[...], 0.0)
        tsc = tsc_ref[...]
        tails_ref[...] = jnp.sum(xt, axis=1, keepdims=True)
        tailpt_ref[...] = jnp.sum(
            jnp.where(colt == tsc, xt, 0.0), axis=1, keepdims=True)

    t = t_ref[...]
    base = i * _BV

    def _accumulate(masked):
        for k in range(_BV // _LW):
            x = p_ref[:, k * _LW:(k + 1) * _LW]
            col = (base + k * _LW) + jax.lax.broadcasted_iota(
                jnp.int32, (n_rows, _LW), 1)
            if masked:
                x = jnp.where(col < vocab, x, 0.0)
            acc_ref[...] += x
            pt_ref[...] += jnp.where(col == t, x, 0.0)

    @pl.when(i < nb - 1)
    def _main():
        _accumulate(masked=False)

    @pl.when(i == nb - 1)
    def _last():
        _accumulate(masked=True)
        eps = _SMOOTH / (vocab - 2)
        conf = 1.0 - _SMOOTH
        kconst = conf * math.log(conf) + (vocab - 2) * eps * math.log(eps)
        s = jnp.sum(acc_ref[...], axis=1, keepdims=True)
        pt = jnp.sum(pt_ref[...], axis=1, keepdims=True)
        p0 = p0_ref[...]
        row = jnp.float32(kconst) - jnp.float32(eps) * (s - p0 - pt) \
            - jnp.float32(conf) * pt
        masked_row = jnp.where(t != _PAD, row, 0.0)
        out_ref[...] = jnp.sum(masked_row, axis=0, keepdims=True)


def _combine_body(tcpart_ref, sums_ref, scpt_ref, scp0_ref,
                  tails_ref, tailpt_ref, t_ref, out_ref, *, vocab):
    eps = _SMOOTH / (vocab - 2)
    conf = 1.0 - _SMOOTH
    kconst = conf * math.log(conf) + (vocab - 2) * eps * math.log(eps)
    s = jnp.sum(sums_ref[...], axis=1, keepdims=True) + tails_ref[...]
    pt = jnp.sum(scpt_ref[...], axis=1, keepdims=True) + tailpt_ref[...]
    p0 = jnp.sum(scp0_ref[...], axis=1, keepdims=True)
    t = t_ref[...]
    row = jnp.float32(kconst) - jnp.float32(eps) * (s - p0 - pt) \
        - jnp.float32(conf) * pt
    masked = jnp.where(t != _PAD, row, 0.0)
    out_ref[...] = tcpart_ref[...] + jnp.sum(masked, axis=0, keepdims=True)


def kernel(predicted_target, target):
    n, v = predicted_target.shape
    n_sc = _SC_ROWS
    n_tc = n - n_sc
    rpt = n_sc // _NW
    nb = (v + _BV - 1) // _BV

    t_sc = target[n_tc:]
    in_sc = t_sc < _SC_COLS
    seg_l = jnp.where(in_sc, t_sc // _SEG, 10 ** 6).astype(jnp.int32)
    off_l = jnp.where(in_sc, t_sc % _SEG, -1).astype(jnp.int32)
    segs = jnp.broadcast_to(seg_l.reshape(_NW, rpt, 1), (_NW, rpt, 16))
    offs = jnp.broadcast_to(off_l.reshape(_NW, rpt, 1), (_NW, rpt, 16))

    sc_sums, sc_pt, sc_p0 = _sc_rowsum_kernel(
        predicted_target, segs, offs, n_tc=n_tc, vocab=v)

    tc_part, tail_sums, tail_pt = pl.pallas_call(
        functools.partial(_tc_body, n_rows=n_tc, n_sc=n_sc, vocab=v),
        grid=(nb,),
        in_specs=[
            pl.BlockSpec((n_tc, _BV), lambda i: (0, i)),
            pl.BlockSpec((n_sc, 128),
                         lambda i, _r=n_tc // n_sc, _c=_SC_COLS // 128:
                         (_r, _c)),
            pl.BlockSpec((n_tc, 1), lambda i: (0, 0)),
            pl.BlockSpec((n_sc, 1), lambda i: (0, 0)),
        ],
        out_specs=[
            pl.BlockSpec((1, 1), lambda i: (0, 0)),
            pl.BlockSpec((n_sc, 1), lambda i: (0, 0)),
            pl.BlockSpec((n_sc, 1), lambda i: (0, 0)),
        ],
        out_shape=[
            jax.ShapeDtypeStruct((1, 1), jnp.float32),
            jax.ShapeDtypeStruct((n_sc, 1), jnp.float32),
            jax.ShapeDtypeStruct((n_sc, 1), jnp.float32),
        ],
        scratch_shapes=[
            pltpu.VMEM((n_tc, _LW), jnp.float32),
            pltpu.VMEM((n_tc, _LW), jnp.float32),
            pltpu.VMEM((n_tc, 1), jnp.float32),
        ],
        compiler_params=pltpu.CompilerParams(
            dimension_semantics=("arbitrary",)),
    )(predicted_target, predicted_target,
      target[:n_tc].reshape(n_tc, 1), t_sc.reshape(n_sc, 1))

    out = pl.pallas_call(
        functools.partial(_combine_body, vocab=v),
        out_shape=jax.ShapeDtypeStruct((1, 1), jnp.float32),
    )(tc_part, sc_sums, sc_pt, sc_p0, tail_sums, tail_pt,
      t_sc.reshape(n_sc, 1))
    return out[0, 0]
